# Initial kernel scaffold; baseline (speedup 1.0000x reference)
import jax
import jax.numpy as jnp
from jax.experimental import pallas as pl


def kernel(x, edge_index, W1, b1, W2, b2):
    raise NotImplementedError("write your pallas kernel here")


# speedup vs baseline: 24.2143x; 24.2143x over previous
import jax
import jax.numpy as jnp
from jax import lax
from jax.experimental import pallas as pl
from jax.experimental.pallas import tpu as pltpu
from jax.experimental.pallas import tpu_sc as plsc

N_NODES = 10000
N_EDGES = 320000
IN_CH = 128
HID_CH = 128
OUT_CH = 64

NC = 2
NS = 16
NW = NC * NS
NP = 10240
RPS = NP // NS
CH = 125
NCHW = 80
NROWS = N_EDGES // CH

_mesh = plsc.VectorSubcoreMesh(core_axis_name="c", subcore_axis_name="s",
                               num_cores=NC, num_subcores=NS)


def _deg_body(dst2_hbm, ones_hbm, zero_hbm, degp_hbm, dstv, onesv, acc, sem):
    c = lax.axis_index("c")
    s = lax.axis_index("s")
    wid = c * NS + s
    base = pl.multiple_of(wid * NCHW, 8)
    zoff = pl.multiple_of(s * RPS, 8)

    pltpu.sync_copy(zero_hbm, acc.at[pl.ds(zoff, RPS)])
    pltpu.sync_copy(ones_hbm, onesv)
    pltpu.sync_copy(dst2_hbm.at[pl.ds(base, NCHW)], dstv)
    plsc.subcore_barrier()

    def step(j, carry):
        pltpu.sync_copy(onesv, acc.at[dstv.at[j]], add=True)
        return carry

    lax.fori_loop(0, NCHW, step, 0)
    plsc.subcore_barrier()
    pltpu.sync_copy(acc.at[pl.ds(zoff, RPS)], degp_hbm.at[c, pl.ds(zoff, RPS)])


def _sc_degree(dst2):
    ones = jnp.ones((CH, 16), dtype=jnp.float32)
    zero = jnp.zeros((RPS, 16), dtype=jnp.float32)
    call = pl.kernel(
        _deg_body,
        out_type=jax.ShapeDtypeStruct((NC, NP, 16), jnp.float32),
        mesh=_mesh,
        compiler_params=pltpu.CompilerParams(use_tc_tiling_on_sc=False),
        scratch_types=[
            pltpu.VMEM((NCHW, CH), jnp.int32),
            pltpu.VMEM((CH, 16), jnp.float32),
            pltpu.VMEM_SHARED((NP, 16), jnp.float32),
            pltpu.SemaphoreType.DMA,
        ],
    )
    return call(dst2, ones, zero)


def _seg_body(src2_hbm, dst2_hbm, g_hbm, zero_hbm, outp_hbm,
              srcv, dstv, rows, acc, sem):
    c = lax.axis_index("c")
    s = lax.axis_index("s")
    wid = c * NS + s
    base = pl.multiple_of(wid * NCHW, 8)
    zoff = pl.multiple_of(s * RPS, 8)

    pltpu.sync_copy(zero_hbm, acc.at[pl.ds(zoff, RPS)])
    pltpu.sync_copy(src2_hbm.at[pl.ds(base, NCHW)], srcv)
    pltpu.sync_copy(dst2_hbm.at[pl.ds(base, NCHW)], dstv)
    plsc.subcore_barrier()

    def step(j, carry):
        pltpu.async_copy(g_hbm.at[srcv.at[j]], rows, sem).wait()
        pltpu.sync_copy(rows, acc.at[dstv.at[j]], add=True)
        return carry

    lax.fori_loop(0, NCHW, step, 0)
    plsc.subcore_barrier()
    pltpu.sync_copy(acc.at[pl.ds(zoff, RPS)], outp_hbm.at[c, pl.ds(zoff, RPS)])


def _sc_segsum(src2, dst2, g, d):
    zero = jnp.zeros((RPS, d), dtype=jnp.float32)
    cp = None if d % 128 == 0 else pltpu.CompilerParams(use_tc_tiling_on_sc=False)
    call = pl.kernel(
        _seg_body,
        out_type=jax.ShapeDtypeStruct((NC, NP, d), jnp.float32),
        mesh=_mesh,
        compiler_params=cp,
        scratch_types=[
            pltpu.VMEM((NCHW, CH), jnp.int32),
            pltpu.VMEM((NCHW, CH), jnp.int32),
            pltpu.VMEM((CH, d), jnp.float32),
            pltpu.VMEM_SHARED((NP, d), jnp.float32),
            pltpu.SemaphoreType.DMA,
        ],
    )
    return call(src2, dst2, g, zero)


_BR = 1000


def _mm_body(x_ref, w_ref, o_ref):
    o_ref[...] = jnp.dot(x_ref[...], w_ref[...],
                         preferred_element_type=jnp.float32)


def _tc_matmul(x, w):
    m, k = x.shape
    n = w.shape[1]
    return pl.pallas_call(
        _mm_body,
        grid=(m // _BR,),
        in_specs=[
            pl.BlockSpec((_BR, k), lambda i: (i, 0)),
            pl.BlockSpec((k, n), lambda i: (0, 0)),
        ],
        out_specs=pl.BlockSpec((_BR, n), lambda i: (i, 0)),
        out_shape=jax.ShapeDtypeStruct((m, n), jnp.float32),
    )(x, w)


def _scale_body(degp_ref, h_ref, g_ref, dis_ref):
    deg = degp_ref[0, :, 0] + degp_ref[1, :, 0] + 1.0
    dis = lax.rsqrt(deg)
    dis_ref[...] = dis[:, None]
    g_ref[...] = h_ref[...] * dis[:, None]


def _tc_scale(degp, h):
    n, d = h.shape
    return pl.pallas_call(
        _scale_body,
        grid=(n // _BR,),
        in_specs=[
            pl.BlockSpec((2, _BR, 16), lambda i: (0, i, 0)),
            pl.BlockSpec((_BR, d), lambda i: (i, 0)),
        ],
        out_specs=[
            pl.BlockSpec((_BR, d), lambda i: (i, 0)),
            pl.BlockSpec((_BR, 1), lambda i: (i, 0)),
        ],
        out_shape=[
            jax.ShapeDtypeStruct((n, d), jnp.float32),
            jax.ShapeDtypeStruct((n, 1), jnp.float32),
        ],
    )(degp, h)


def _mid_body(accp_ref, g1_ref, dis_ref, b1_ref, w2_ref, g2_ref):
    acc = accp_ref[0] + accp_ref[1] + g1_ref[...]
    out1 = jnp.tanh(acc * dis_ref[...] + b1_ref[...])
    h2 = jnp.dot(out1, w2_ref[...], preferred_element_type=jnp.float32)
    g2_ref[...] = h2 * dis_ref[...]


def _tc_mid(accp, g1, dis, b1, w2):
    n, d = g1.shape
    d2 = w2.shape[1]
    return pl.pallas_call(
        _mid_body,
        grid=(n // _BR,),
        in_specs=[
            pl.BlockSpec((2, _BR, d), lambda i: (0, i, 0)),
            pl.BlockSpec((_BR, d), lambda i: (i, 0)),
            pl.BlockSpec((_BR, 1), lambda i: (i, 0)),
            pl.BlockSpec((1, d), lambda i: (0, 0)),
            pl.BlockSpec((d, d2), lambda i: (0, 0)),
        ],
        out_specs=pl.BlockSpec((_BR, d2), lambda i: (i, 0)),
        out_shape=jax.ShapeDtypeStruct((n, d2), jnp.float32),
    )(accp, g1, dis, b1, w2)


def _fin_body(accp_ref, g2_ref, dis_ref, b2_ref, o_ref):
    acc = accp_ref[0] + accp_ref[1] + g2_ref[...]
    o_ref[...] = acc * dis_ref[...] + b2_ref[...]


def _tc_fin(accp, g2, dis, b2):
    n, d = g2.shape
    return pl.pallas_call(
        _fin_body,
        grid=(n // _BR,),
        in_specs=[
            pl.BlockSpec((2, _BR, d), lambda i: (0, i, 0)),
            pl.BlockSpec((_BR, d), lambda i: (i, 0)),
            pl.BlockSpec((_BR, 1), lambda i: (i, 0)),
            pl.BlockSpec((1, d), lambda i: (0, 0)),
        ],
        out_specs=pl.BlockSpec((_BR, d), lambda i: (i, 0)),
        out_shape=jax.ShapeDtypeStruct((n, d), jnp.float32),
    )(accp, g2, dis, b2)


def kernel(x, edge_index, W1, b1, W2, b2):
    src = edge_index[0].astype(jnp.int32)
    dst = edge_index[1].astype(jnp.int32)
    src2 = src.reshape(NROWS, CH)
    dst2 = dst.reshape(NROWS, CH)

    degp = _sc_degree(dst2)
    h1 = _tc_matmul(x, W1)
    g1, dis = _tc_scale(degp, h1)

    accp1 = _sc_segsum(src2, dst2, g1, HID_CH)
    g2 = _tc_mid(accp1, g1, dis, b1.reshape(1, HID_CH), W2)

    accp2 = _sc_segsum(src2, dst2, g2, OUT_CH)
    out = _tc_fin(accp2, g2, dis, b2.reshape(1, OUT_CH))
    return out
